# Initial kernel scaffold; baseline (speedup 1.0000x reference)
import jax
import jax.numpy as jnp
from jax.experimental import pallas as pl


def kernel(input1, input2, W, b):
    raise NotImplementedError("write your pallas kernel here")


# speedup vs baseline: 3.5818x; 3.5818x over previous
import functools

import jax
import jax.numpy as jnp
from jax.experimental import pallas as pl
from jax.experimental.pallas import tpu as pltpu


def _window_sums(x):
    h, w = x.shape[1], x.shape[2]
    colsum = jnp.sum(x, axis=1)
    r0 = x[:, 0, :]
    r1 = x[:, 1, :]
    rm2 = x[:, h - 2, :]
    rm1 = x[:, h - 1, :]
    ps = (colsum - rm2 - rm1, colsum - r0 - rm1, colsum - r0 - r1)
    outs = []
    for p in ps:
        tot = jnp.sum(p, axis=1, keepdims=True)
        c0 = p[:, 0:1]
        c1 = p[:, 1:2]
        cm2 = p[:, w - 2:w - 1]
        cm1 = p[:, w - 1:w]
        outs.append(tot - cm2 - cm1)
        outs.append(tot - c0 - cm1)
        outs.append(tot - c0 - c1)
    return jnp.concatenate(outs, axis=1)


def _sums_kernel(x1_ref, x2_ref, s1_ref, s2_ref, s3_ref):
    x1 = x1_ref[...]
    x2 = x2_ref[...]
    s1_ref[...] = _window_sums(x1)
    s2_ref[...] = _window_sums(x2)
    s3_ref[...] = _window_sums(x1 + x2)


def _coef_kernel(s_ref, wt_ref, b_ref, c1_ref, c2_ref, *, inv_n):
    m = jnp.dot(s_ref[...], wt_ref[...], preferred_element_type=jnp.float32)
    m = m * inv_n + b_ref[...]
    x = jax.nn.sigmoid(m)
    x1 = x[0:1, :]
    x2 = x[1:2, :]
    x3 = x[2:3, :]
    c1 = x1 >= x2
    c2 = x1 <= x2
    a1 = c1 & (x1 >= x3)
    a2 = c2 & (x2 >= x3)
    a3 = (c1 & (x1 < x3)) | (c2 & (x2 < x3))
    f3 = jnp.where(a3, 1.0, 0.0)
    c1_ref[...] = jnp.where(a1, 1.0, 0.0) + f3
    c2_ref[...] = jnp.where(a2, 1.0, 0.0) + f3


def _combine_kernel(x1_ref, x2_ref, c1_ref, c2_ref, o_ref):
    o_ref[...] = c1_ref[...] * x1_ref[...] + c2_ref[...] * x2_ref[...]


def kernel(input1, input2, W, b):
    _, C, H, W_sp = input1.shape
    HW = H * W_sp
    K = 9 * C
    n_valid = (H - 2) * (W_sp - 2)

    x1 = input1.reshape(C, H, W_sp)
    x2 = input2.reshape(C, H, W_sp)

    CB = 8
    s1, s2, s3 = pl.pallas_call(
        _sums_kernel,
        grid=(C // CB,),
        in_specs=[
            pl.BlockSpec((CB, H, W_sp), lambda i: (i, 0, 0)),
            pl.BlockSpec((CB, H, W_sp), lambda i: (i, 0, 0)),
        ],
        out_specs=[
            pl.BlockSpec((CB, 9), lambda i: (i, 0)),
            pl.BlockSpec((CB, 9), lambda i: (i, 0)),
            pl.BlockSpec((CB, 9), lambda i: (i, 0)),
        ],
        out_shape=[jax.ShapeDtypeStruct((C, 9), jnp.float32)] * 3,
        compiler_params=pltpu.CompilerParams(
            dimension_semantics=("arbitrary",),
            vmem_limit_bytes=56 * 1024 * 1024,
        ),
        name="gkdm_window_sums",
    )(x1, x2)

    smat = jnp.concatenate(
        [s1.reshape(1, K), s2.reshape(1, K), s3.reshape(1, K)], axis=0
    )
    wt = W.transpose(1, 2, 3, 0).reshape(K, C)
    b_row = b.reshape(1, C)

    coef1, coef2 = pl.pallas_call(
        functools.partial(_coef_kernel, inv_n=1.0 / n_valid),
        in_specs=[
            pl.BlockSpec((3, K), lambda: (0, 0)),
            pl.BlockSpec((K, C), lambda: (0, 0)),
            pl.BlockSpec((1, C), lambda: (0, 0)),
        ],
        out_specs=[
            pl.BlockSpec((1, C), lambda: (0, 0)),
            pl.BlockSpec((1, C), lambda: (0, 0)),
        ],
        out_shape=[jax.ShapeDtypeStruct((1, C), jnp.float32)] * 2,
        compiler_params=pltpu.CompilerParams(
            vmem_limit_bytes=56 * 1024 * 1024,
        ),
        name="gkdm_coefs",
    )(smat, wt, b_row)

    x1v = input1.reshape(C, HW)
    x2v = input2.reshape(C, HW)
    out2d = pl.pallas_call(
        _combine_kernel,
        grid=(C // CB,),
        in_specs=[
            pl.BlockSpec((CB, HW), lambda i: (i, 0)),
            pl.BlockSpec((CB, HW), lambda i: (i, 0)),
            pl.BlockSpec((CB, 1), lambda i: (i, 0)),
            pl.BlockSpec((CB, 1), lambda i: (i, 0)),
        ],
        out_specs=pl.BlockSpec((CB, HW), lambda i: (i, 0)),
        out_shape=jax.ShapeDtypeStruct((C, HW), jnp.float32),
        compiler_params=pltpu.CompilerParams(
            dimension_semantics=("arbitrary",),
            vmem_limit_bytes=56 * 1024 * 1024,
        ),
        name="gkdm_combine",
    )(x1v, x2v, coef1.reshape(C, 1), coef2.reshape(C, 1))

    return out2d.reshape(1, C, H, W_sp)


# speedup vs baseline: 8.1587x; 2.2778x over previous
import functools

import jax
import jax.numpy as jnp
from jax.experimental import pallas as pl
from jax.experimental.pallas import tpu as pltpu


def _window_sums(x):
    h, w = x.shape[1], x.shape[2]
    colsum = jnp.sum(x, axis=1)
    r0 = x[:, 0, :]
    r1 = x[:, 1, :]
    rm2 = x[:, h - 2, :]
    rm1 = x[:, h - 1, :]
    ps = (colsum - rm2 - rm1, colsum - r0 - rm1, colsum - r0 - r1)
    outs = []
    for p in ps:
        tot = jnp.sum(p, axis=1, keepdims=True)
        c0 = p[:, 0:1]
        c1 = p[:, 1:2]
        cm2 = p[:, w - 2:w - 1]
        cm1 = p[:, w - 1:w]
        outs.append(tot - cm2 - cm1)
        outs.append(tot - c0 - cm1)
        outs.append(tot - c0 - c1)
    return jnp.concatenate(outs, axis=1)


def _sums_kernel(x1_ref, x2_ref, s1_ref, s2_ref, s3_ref):
    x1 = x1_ref[...]
    x2 = x2_ref[...]
    s1_ref[...] = _window_sums(x1)
    s2_ref[...] = _window_sums(x2)
    s3_ref[...] = _window_sums(x1 + x2)


def _coef_kernel(s_ref, wt_ref, b_ref, c1_ref, c2_ref, *, inv_n):
    m = jnp.dot(s_ref[...], wt_ref[...], preferred_element_type=jnp.float32)
    m = m * inv_n + b_ref[...]
    x = jax.nn.sigmoid(m)
    x1 = x[0:1, :]
    x2 = x[1:2, :]
    x3 = x[2:3, :]
    c1 = x1 >= x2
    c2 = x1 <= x2
    a1 = c1 & (x1 >= x3)
    a2 = c2 & (x2 >= x3)
    a3 = (c1 & (x1 < x3)) | (c2 & (x2 < x3))
    f3 = jnp.where(a3, 1.0, 0.0)
    c1_ref[...] = jnp.where(a1, 1.0, 0.0) + f3
    c2_ref[...] = jnp.where(a2, 1.0, 0.0) + f3


def _combine_kernel(x1_ref, x2_ref, c1_ref, c2_ref, o_ref):
    o_ref[...] = c1_ref[...] * x1_ref[...] + c2_ref[...] * x2_ref[...]


def kernel(input1, input2, W, b):
    _, C, H, W_sp = input1.shape
    HW = H * W_sp
    K = 9 * C
    n_valid = (H - 2) * (W_sp - 2)

    x1 = input1.reshape(C, H, W_sp)
    x2 = input2.reshape(C, H, W_sp)

    CB = 8
    s1, s2, s3 = pl.pallas_call(
        _sums_kernel,
        grid=(C // CB,),
        in_specs=[
            pl.BlockSpec((CB, H, W_sp), lambda i: (i, 0, 0)),
            pl.BlockSpec((CB, H, W_sp), lambda i: (i, 0, 0)),
        ],
        out_specs=[
            pl.BlockSpec((CB, 9), lambda i: (i, 0)),
            pl.BlockSpec((CB, 9), lambda i: (i, 0)),
            pl.BlockSpec((CB, 9), lambda i: (i, 0)),
        ],
        out_shape=[jax.ShapeDtypeStruct((C, 9), jnp.float32)] * 3,
        compiler_params=pltpu.CompilerParams(
            dimension_semantics=("arbitrary",),
            vmem_limit_bytes=56 * 1024 * 1024,
        ),
        name="gkdm_window_sums",
    )(x1, x2)

    smat = jnp.concatenate(
        [s1.reshape(1, K), s2.reshape(1, K), s3.reshape(1, K)], axis=0
    )
    wt = W.transpose(1, 2, 3, 0).reshape(K, C)
    b_row = b.reshape(1, C)

    coef1, coef2 = pl.pallas_call(
        functools.partial(_coef_kernel, inv_n=1.0 / n_valid),
        in_specs=[
            pl.BlockSpec((3, K), lambda: (0, 0)),
            pl.BlockSpec((K, C), lambda: (0, 0)),
            pl.BlockSpec((1, C), lambda: (0, 0)),
        ],
        out_specs=[
            pl.BlockSpec((1, C), lambda: (0, 0)),
            pl.BlockSpec((1, C), lambda: (0, 0)),
        ],
        out_shape=[jax.ShapeDtypeStruct((1, C), jnp.float32)] * 2,
        compiler_params=pltpu.CompilerParams(
            vmem_limit_bytes=56 * 1024 * 1024,
        ),
        name="gkdm_coefs",
    )(smat, wt, b_row)

    out3d = pl.pallas_call(
        _combine_kernel,
        grid=(C // CB,),
        in_specs=[
            pl.BlockSpec((CB, H, W_sp), lambda i: (i, 0, 0)),
            pl.BlockSpec((CB, H, W_sp), lambda i: (i, 0, 0)),
            pl.BlockSpec((CB, 1, 1), lambda i: (i, 0, 0)),
            pl.BlockSpec((CB, 1, 1), lambda i: (i, 0, 0)),
        ],
        out_specs=pl.BlockSpec((CB, H, W_sp), lambda i: (i, 0, 0)),
        out_shape=jax.ShapeDtypeStruct((C, H, W_sp), jnp.float32),
        compiler_params=pltpu.CompilerParams(
            dimension_semantics=("arbitrary",),
            vmem_limit_bytes=56 * 1024 * 1024,
        ),
        name="gkdm_combine",
    )(x1, x2, coef1.reshape(C, 1, 1), coef2.reshape(C, 1, 1))

    return out3d.reshape(1, C, H, W_sp)


# speedup vs baseline: 8.2059x; 1.0058x over previous
import functools

import jax
import jax.numpy as jnp
from jax.experimental import pallas as pl
from jax.experimental.pallas import tpu as pltpu


def _window_sums(x):
    h, w = x.shape[1], x.shape[2]
    colsum = jnp.sum(x, axis=1)
    r0 = x[:, 0, :]
    r1 = x[:, 1, :]
    rm2 = x[:, h - 2, :]
    rm1 = x[:, h - 1, :]
    ps = (colsum - rm2 - rm1, colsum - r0 - rm1, colsum - r0 - r1)
    outs = []
    for p in ps:
        tot = jnp.sum(p, axis=1, keepdims=True)
        c0 = p[:, 0:1]
        c1 = p[:, 1:2]
        cm2 = p[:, w - 2:w - 1]
        cm1 = p[:, w - 1:w]
        outs.append(tot - cm2 - cm1)
        outs.append(tot - c0 - cm1)
        outs.append(tot - c0 - c1)
    return jnp.concatenate(outs, axis=1)


def _sums_kernel(x1_ref, x2_ref, s1_ref, s2_ref, s3_ref):
    x1 = x1_ref[...]
    x2 = x2_ref[...]
    s1_ref[...] = _window_sums(x1)
    s2_ref[...] = _window_sums(x2)
    s3_ref[...] = _window_sums(x1 + x2)


def _coef_kernel(s_ref, wt_ref, b_ref, c1_ref, c2_ref, *, inv_n):
    m = jnp.dot(s_ref[...], wt_ref[...], preferred_element_type=jnp.float32)
    m = m * inv_n + b_ref[...]
    x = jax.nn.sigmoid(m)
    x1 = x[0:1, :]
    x2 = x[1:2, :]
    x3 = x[2:3, :]
    c1 = x1 >= x2
    c2 = x1 <= x2
    a1 = c1 & (x1 >= x3)
    a2 = c2 & (x2 >= x3)
    a3 = (c1 & (x1 < x3)) | (c2 & (x2 < x3))
    f3 = jnp.where(a3, 1.0, 0.0)
    c1_ref[...] = jnp.where(a1, 1.0, 0.0) + f3
    c2_ref[...] = jnp.where(a2, 1.0, 0.0) + f3


def _combine_kernel(x1_ref, x2_ref, c1_ref, c2_ref, o_ref):
    o_ref[...] = c1_ref[...] * x1_ref[...] + c2_ref[...] * x2_ref[...]


def kernel(input1, input2, W, b):
    _, C, H, W_sp = input1.shape
    HW = H * W_sp
    K = 9 * C
    n_valid = (H - 2) * (W_sp - 2)

    x1 = input1.reshape(C, H, W_sp)
    x2 = input2.reshape(C, H, W_sp)

    CB = 8
    CBA = 16
    s1, s2, s3 = pl.pallas_call(
        _sums_kernel,
        grid=(C // CBA,),
        in_specs=[
            pl.BlockSpec((CBA, H, W_sp), lambda i: (i, 0, 0)),
            pl.BlockSpec((CBA, H, W_sp), lambda i: (i, 0, 0)),
        ],
        out_specs=[
            pl.BlockSpec((CBA, 9), lambda i: (i, 0)),
            pl.BlockSpec((CBA, 9), lambda i: (i, 0)),
            pl.BlockSpec((CBA, 9), lambda i: (i, 0)),
        ],
        out_shape=[jax.ShapeDtypeStruct((C, 9), jnp.float32)] * 3,
        compiler_params=pltpu.CompilerParams(
            dimension_semantics=("arbitrary",),
            vmem_limit_bytes=56 * 1024 * 1024,
        ),
        name="gkdm_window_sums",
    )(x1, x2)

    smat = jnp.concatenate(
        [s1.reshape(1, K), s2.reshape(1, K), s3.reshape(1, K)], axis=0
    )
    wt = W.transpose(1, 2, 3, 0).reshape(K, C)
    b_row = b.reshape(1, C)

    coef1, coef2 = pl.pallas_call(
        functools.partial(_coef_kernel, inv_n=1.0 / n_valid),
        in_specs=[
            pl.BlockSpec((3, K), lambda: (0, 0)),
            pl.BlockSpec((K, C), lambda: (0, 0)),
            pl.BlockSpec((1, C), lambda: (0, 0)),
        ],
        out_specs=[
            pl.BlockSpec((1, C), lambda: (0, 0)),
            pl.BlockSpec((1, C), lambda: (0, 0)),
        ],
        out_shape=[jax.ShapeDtypeStruct((1, C), jnp.float32)] * 2,
        compiler_params=pltpu.CompilerParams(
            vmem_limit_bytes=56 * 1024 * 1024,
        ),
        name="gkdm_coefs",
    )(smat, wt, b_row)

    out3d = pl.pallas_call(
        _combine_kernel,
        grid=(C // CB,),
        in_specs=[
            pl.BlockSpec((CB, H, W_sp), lambda i: (i, 0, 0)),
            pl.BlockSpec((CB, H, W_sp), lambda i: (i, 0, 0)),
            pl.BlockSpec((CB, 1, 1), lambda i: (i, 0, 0)),
            pl.BlockSpec((CB, 1, 1), lambda i: (i, 0, 0)),
        ],
        out_specs=pl.BlockSpec((CB, H, W_sp), lambda i: (i, 0, 0)),
        out_shape=jax.ShapeDtypeStruct((C, H, W_sp), jnp.float32),
        compiler_params=pltpu.CompilerParams(
            dimension_semantics=("arbitrary",),
            vmem_limit_bytes=56 * 1024 * 1024,
        ),
        name="gkdm_combine",
    )(x1, x2, coef1.reshape(C, 1, 1), coef2.reshape(C, 1, 1))

    return out3d.reshape(1, C, H, W_sp)


# speedup vs baseline: 9.2166x; 1.1232x over previous
import functools

import jax
import jax.numpy as jnp
from jax.experimental import pallas as pl
from jax.experimental.pallas import tpu as pltpu


def _window_sums(x):
    h, w = x.shape[1], x.shape[2]
    colsum = jnp.sum(x, axis=1)
    r0 = x[:, 0, :]
    r1 = x[:, 1, :]
    rm2 = x[:, h - 2, :]
    rm1 = x[:, h - 1, :]
    ps = (colsum - rm2 - rm1, colsum - r0 - rm1, colsum - r0 - r1)
    outs = []
    for p in ps:
        tot = jnp.sum(p, axis=1, keepdims=True)
        c0 = p[:, 0:1]
        c1 = p[:, 1:2]
        cm2 = p[:, w - 2:w - 1]
        cm1 = p[:, w - 1:w]
        outs.append(tot - cm2 - cm1)
        outs.append(tot - c0 - cm1)
        outs.append(tot - c0 - c1)
    return jnp.concatenate(outs, axis=1)


def _sums_kernel(x1_ref, x2_ref, s1_ref, s2_ref, s3_ref):
    x1 = x1_ref[...]
    x2 = x2_ref[...]
    s1_ref[...] = _window_sums(x1)
    s2_ref[...] = _window_sums(x2)
    s3_ref[...] = _window_sums(x1 + x2)


def _coef_kernel(s_ref, wt_ref, b_ref, c1_ref, c2_ref, *, inv_n):
    m = jnp.dot(s_ref[...], wt_ref[...], preferred_element_type=jnp.float32)
    m = m * inv_n + b_ref[...]
    x = jax.nn.sigmoid(m)
    x1 = x[0:1, :]
    x2 = x[1:2, :]
    x3 = x[2:3, :]
    c1 = x1 >= x2
    c2 = x1 <= x2
    a1 = c1 & (x1 >= x3)
    a2 = c2 & (x2 >= x3)
    a3 = (c1 & (x1 < x3)) | (c2 & (x2 < x3))
    f3 = jnp.where(a3, 1.0, 0.0)
    c1_ref[...] = jnp.where(a1, 1.0, 0.0) + f3
    c2_ref[...] = jnp.where(a2, 1.0, 0.0) + f3


def _combine_kernel(c1s_ref, c2s_ref, x1_hbm, x2_hbm, c1v_ref, c2v_ref,
                    o_ref, buf1, buf2, sem1, sem2, *, cb):
    i = pl.program_id(0)
    n = pl.num_programs(0)
    slot = jax.lax.rem(i, 2)
    nxt = jax.lax.rem(i + 1, 2)

    def issue(step, sl):
        for k in range(cb):
            ch = step * cb + k

            @pl.when(c1s_ref[ch] != 0)
            def _():
                pltpu.make_async_copy(
                    x1_hbm.at[ch], buf1.at[sl, k], sem1.at[sl, k]).start()

            @pl.when(c2s_ref[ch] != 0)
            def _():
                pltpu.make_async_copy(
                    x2_hbm.at[ch], buf2.at[sl, k], sem2.at[sl, k]).start()

    @pl.when(i == 0)
    def _():
        issue(i, slot)

    @pl.when(i + 1 < n)
    def _():
        issue(i + 1, nxt)

    for k in range(cb):
        ch = i * cb + k

        @pl.when(c1s_ref[ch] != 0)
        def _():
            pltpu.make_async_copy(
                x1_hbm.at[ch], buf1.at[slot, k], sem1.at[slot, k]).wait()

        @pl.when(c2s_ref[ch] != 0)
        def _():
            pltpu.make_async_copy(
                x2_hbm.at[ch], buf2.at[slot, k], sem2.at[slot, k]).wait()

    c1 = c1v_ref[...]
    c2 = c2v_ref[...]
    x1b = buf1[slot]
    x2b = buf2[slot]
    o_ref[...] = jnp.where(c1 == 0.0, x2b, jnp.where(c2 == 0.0, x1b, x1b + x2b))


def kernel(input1, input2, W, b):
    _, C, H, W_sp = input1.shape
    HW = H * W_sp
    K = 9 * C
    n_valid = (H - 2) * (W_sp - 2)

    x1 = input1.reshape(C, H, W_sp)
    x2 = input2.reshape(C, H, W_sp)

    CB = 8
    CBA = 16
    s1, s2, s3 = pl.pallas_call(
        _sums_kernel,
        grid=(C // CBA,),
        in_specs=[
            pl.BlockSpec((CBA, H, W_sp), lambda i: (i, 0, 0)),
            pl.BlockSpec((CBA, H, W_sp), lambda i: (i, 0, 0)),
        ],
        out_specs=[
            pl.BlockSpec((CBA, 9), lambda i: (i, 0)),
            pl.BlockSpec((CBA, 9), lambda i: (i, 0)),
            pl.BlockSpec((CBA, 9), lambda i: (i, 0)),
        ],
        out_shape=[jax.ShapeDtypeStruct((C, 9), jnp.float32)] * 3,
        compiler_params=pltpu.CompilerParams(
            dimension_semantics=("arbitrary",),
            vmem_limit_bytes=56 * 1024 * 1024,
        ),
        name="gkdm_window_sums",
    )(x1, x2)

    smat = jnp.concatenate(
        [s1.reshape(1, K), s2.reshape(1, K), s3.reshape(1, K)], axis=0
    )
    wt = W.transpose(1, 2, 3, 0).reshape(K, C)
    b_row = b.reshape(1, C)

    coef1, coef2 = pl.pallas_call(
        functools.partial(_coef_kernel, inv_n=1.0 / n_valid),
        in_specs=[
            pl.BlockSpec((3, K), lambda: (0, 0)),
            pl.BlockSpec((K, C), lambda: (0, 0)),
            pl.BlockSpec((1, C), lambda: (0, 0)),
        ],
        out_specs=[
            pl.BlockSpec((1, C), lambda: (0, 0)),
            pl.BlockSpec((1, C), lambda: (0, 0)),
        ],
        out_shape=[jax.ShapeDtypeStruct((1, C), jnp.float32)] * 2,
        compiler_params=pltpu.CompilerParams(
            vmem_limit_bytes=56 * 1024 * 1024,
        ),
        name="gkdm_coefs",
    )(smat, wt, b_row)

    c1nz = (coef1[0] != 0).astype(jnp.int32)
    c2nz = (coef2[0] != 0).astype(jnp.int32)
    out3d = pl.pallas_call(
        functools.partial(_combine_kernel, cb=CB),
        grid_spec=pltpu.PrefetchScalarGridSpec(
            num_scalar_prefetch=2,
            grid=(C // CB,),
            in_specs=[
                pl.BlockSpec(memory_space=pl.ANY),
                pl.BlockSpec(memory_space=pl.ANY),
                pl.BlockSpec((CB, 1, 1), lambda i, *_: (i, 0, 0)),
                pl.BlockSpec((CB, 1, 1), lambda i, *_: (i, 0, 0)),
            ],
            out_specs=pl.BlockSpec((CB, H, W_sp), lambda i, *_: (i, 0, 0)),
            scratch_shapes=[
                pltpu.VMEM((2, CB, H, W_sp), jnp.float32),
                pltpu.VMEM((2, CB, H, W_sp), jnp.float32),
                pltpu.SemaphoreType.DMA((2, CB)),
                pltpu.SemaphoreType.DMA((2, CB)),
            ],
        ),
        out_shape=jax.ShapeDtypeStruct((C, H, W_sp), jnp.float32),
        compiler_params=pltpu.CompilerParams(
            dimension_semantics=("arbitrary",),
            vmem_limit_bytes=56 * 1024 * 1024,
        ),
        name="gkdm_combine",
    )(c1nz, c2nz, x1, x2, coef1.reshape(C, 1, 1), coef2.reshape(C, 1, 1))

    return out3d.reshape(1, C, H, W_sp)
